# Initial kernel scaffold; baseline (speedup 1.0000x reference)
import jax
import jax.numpy as jnp
from jax.experimental import pallas as pl


def kernel(x, W, endW):
    raise NotImplementedError("write your pallas kernel here")


# speedup vs baseline: 121.3154x; 121.3154x over previous
import functools

import jax
import jax.numpy as jnp
from jax import lax
from jax.experimental import pallas as pl
from jax.experimental.pallas import tpu as pltpu
from jax.experimental.pallas import tpu_sc as plsc

N = 100
K = 101
B = 16384
LANE = 128
TBL_ROWS = N
TBL_FLAT = TBL_ROWS * 2 * LANE
NEG = -1e30


def _table_kernel(p_ref, o_ref):
    a = p_ref[:, :LANE]
    b = p_ref[:, LANE:]
    m = jnp.maximum(a, b)
    lse2 = m + jnp.log(jnp.exp(a - m) + jnp.exp(b - m))
    l0 = a - lse2
    l1 = b - lse2
    rmax = jnp.max(a, axis=1, keepdims=True)
    rsum = jnp.sum(jnp.exp(a - rmax), axis=1, keepdims=True)
    le = a - rmax - jnp.log(rsum)
    r = lax.broadcasted_iota(jnp.int32, (TBL_ROWS, LANE), 0)
    g = lax.broadcasted_iota(jnp.int32, (TBL_ROWS, LANE), 1)
    step_mask = (g >= 1) & (g <= r + 1) & (r <= N - 2)
    end_mask = (r == N - 1) & (g <= K - 1)
    o_ref[:, :LANE] = jnp.where(step_mask, l0, jnp.where(end_mask, le, 0.0))
    o_ref[:, LANE:] = jnp.where(step_mask, l1, 0.0)


def _build_table(W, endW):
    c0 = jnp.zeros((TBL_ROWS, LANE), jnp.float32)
    c0 = c0.at[: N - 1, 1:K].set(W[:, :, 0])
    c0 = c0.at[N - 1, :].set(NEG)
    c0 = c0.at[N - 1, :K].set(endW[0, :])
    c1 = jnp.zeros((TBL_ROWS, LANE), jnp.float32)
    c1 = c1.at[: N - 1, 1:K].set(W[:, :, 1])
    p = jnp.concatenate([c0, c1], axis=1)
    return pl.pallas_call(
        _table_kernel,
        out_shape=jax.ShapeDtypeStruct((TBL_ROWS, 2 * LANE), jnp.float32),
    )(p)


def _make_sc_kernel():
    info = plsc.get_sparse_core_info()
    nc, ns = info.num_cores, info.num_subcores
    nw = nc * ns
    bpw = B // nw
    groups = bpw // 16
    mesh = plsc.VectorSubcoreMesh(core_axis_name="c", subcore_axis_name="s")

    @functools.partial(
        pl.kernel,
        mesh=mesh,
        out_type=jax.ShapeDtypeStruct((B,), jnp.float32),
        scratch_types=[
            pltpu.VMEM((bpw * N,), jnp.int32),
            pltpu.VMEM((TBL_FLAT,), jnp.float32),
            pltpu.VMEM((bpw,), jnp.float32),
        ],
        compiler_params=pltpu.CompilerParams(needs_layout_passes=False),
    )
    def sc_fn(x_hbm, tbl_hbm, out_hbm, x_v, tbl_v, out_v):
        wid = lax.axis_index("s") * nc + lax.axis_index("c")
        base = wid * bpw
        pltpu.sync_copy(x_hbm.at[pl.ds(base * N, bpw * N)], x_v)
        pltpu.sync_copy(tbl_hbm, tbl_v)
        lanes = lax.iota(jnp.int32, 16)

        def cbody(c, carry):
            rowoff = (lanes + c * 16) * N
            xv0 = plsc.load_gather(x_v, [rowoff])

            def jbody(j, gc):
                g, acc = gc
                xv = plsc.load_gather(x_v, [rowoff + j])
                g = g + xv
                idx = (j - 1) * 256 + xv * 128 + g
                tv = plsc.load_gather(tbl_v, [idx])
                return (g, acc + tv)

            g, acc = lax.fori_loop(
                1, N, jbody, (xv0, jnp.zeros((16,), jnp.float32))
            )
            acc = acc + plsc.load_gather(tbl_v, [g + (N - 1) * 256])
            out_v[pl.ds(c * 16, 16)] = acc
            return carry

        lax.fori_loop(0, groups, cbody, 0)
        pltpu.sync_copy(out_v, out_hbm.at[pl.ds(base, bpw)])

    return sc_fn


_SC_KERNEL = None


def kernel(x, W, endW):
    global _SC_KERNEL
    if _SC_KERNEL is None:
        _SC_KERNEL = _make_sc_kernel()
    table = _build_table(W, endW)
    out = _SC_KERNEL(x.reshape(-1).astype(jnp.int32), table.reshape(-1))
    return out[:, None]
